# baseline (speedup 1.0000x reference)
import functools

import jax
import jax.numpy as jnp
from jax import lax
from jax.experimental import pallas as pl
from jax.experimental.pallas import tpu as pltpu
from jax.experimental.pallas import tpu_sc as plsc

N_VOCAB_ROWS = 1000000
D_EMBED = 64
D_PAD = 128
B_TOTAL = 4096 * 200
NUM_WORKERS = 32
B_PER_W = B_TOTAL // NUM_WORKERS
CHUNK = 128
N_CHUNK = B_PER_W // CHUNK
NBUF = 5
NG = N_CHUNK // NBUF
assert N_CHUNK % NBUF == 0 and B_PER_W % CHUNK == 0 and B_TOTAL % NUM_WORKERS == 0

PREP_COLS = 8192


def _prep_body(wt_ref, out_ref):
    out_ref[:, 0:D_EMBED] = jnp.transpose(wt_ref[...], (1, 0))


_prep = pl.pallas_call(
    _prep_body,
    grid=(pl.cdiv(N_VOCAB_ROWS, PREP_COLS),),
    in_specs=[pl.BlockSpec((D_EMBED, PREP_COLS), lambda i: (0, i))],
    out_specs=pl.BlockSpec((PREP_COLS, D_PAD), lambda i: (i, 0)),
    out_shape=jax.ShapeDtypeStruct((N_VOCAB_ROWS, D_PAD), jnp.float32),
)


@functools.partial(
    pl.kernel,
    out_type=jax.ShapeDtypeStruct((B_TOTAL, D_PAD), jnp.float32),
    mesh=plsc.VectorSubcoreMesh(core_axis_name="c", subcore_axis_name="s"),
    scratch_types=[
        pltpu.VMEM((B_PER_W,), jnp.int32),
        pltpu.VMEM((NBUF, CHUNK, D_PAD), jnp.float32),
        pltpu.SemaphoreType.DMA((NBUF,)),
        pltpu.SemaphoreType.DMA((NBUF,)),
    ],
)
def _embed_gather(idx_hbm, table_hbm, out_hbm, idx_v, rows_v, gsem, wsem):
    wid = lax.axis_index("s") * 2 + lax.axis_index("c")
    base = wid * B_PER_W

    pltpu.sync_copy(idx_hbm.at[pl.ds(base, B_PER_W)], idx_v)

    def start_gather(b, i):
        pltpu.async_copy(
            table_hbm.at[idx_v.at[pl.ds(i * CHUNK, CHUNK)]],
            rows_v.at[b],
            gsem.at[b],
        )

    def wait_gather(b, i):
        pltpu.make_async_copy(
            table_hbm.at[idx_v.at[pl.ds(i * CHUNK, CHUNK)]],
            rows_v.at[b],
            gsem.at[b],
        ).wait()

    def start_wb(b, i):
        pltpu.async_copy(
            rows_v.at[b],
            out_hbm.at[pl.ds(base + i * CHUNK, CHUNK)],
            wsem.at[b],
        )

    def wait_wb(b, i):
        pltpu.make_async_copy(
            rows_v.at[b],
            out_hbm.at[pl.ds(base + i * CHUNK, CHUNK)],
            wsem.at[b],
        ).wait()

    for b in range(NBUF):
        start_gather(b, b)

    def outer(g, carry):
        for b in range(NBUF):
            i = g * NBUF + b
            wait_gather(b, i)
            start_wb(b, i)
            wait_wb(b, i)
            start_gather(b, i + NBUF)
        return carry

    lax.fori_loop(0, NG - 1, outer, 0)

    for b in range(NBUF):
        i = (NG - 1) * NBUF + b
        wait_gather(b, i)
        start_wb(b, i)
    for b in range(NBUF):
        i = (NG - 1) * NBUF + b
        wait_wb(b, i)


def kernel(x, W_E):
    flat = x.reshape(B_TOTAL).astype(jnp.int32)
    table = _prep(W_E.T)
    out = _embed_gather(flat, table)
    return out[:, :D_EMBED].reshape(x.shape[0], x.shape[1], D_EMBED)


# speedup vs baseline: 1.0271x; 1.0271x over previous
import functools

import jax
import jax.numpy as jnp
from jax import lax
from jax.experimental import pallas as pl
from jax.experimental.pallas import tpu as pltpu
from jax.experimental.pallas import tpu_sc as plsc

N_VOCAB_ROWS = 1000000
D_EMBED = 64
D_PAD = 128
B_TOTAL = 4096 * 200
NUM_WORKERS = 32
B_PER_W = B_TOTAL // NUM_WORKERS
CHUNK = 128
N_CHUNK = B_PER_W // CHUNK
NBUF = 5
NG = N_CHUNK // NBUF
assert N_CHUNK % NBUF == 0 and B_PER_W % CHUNK == 0 and B_TOTAL % NUM_WORKERS == 0

PREP_COLS = 16384


def _prep_body(wt_ref, out_ref):
    out_ref[:, 0:D_EMBED] = jnp.transpose(wt_ref[...], (1, 0))


_prep = pl.pallas_call(
    _prep_body,
    grid=(pl.cdiv(N_VOCAB_ROWS, PREP_COLS),),
    in_specs=[pl.BlockSpec((D_EMBED, PREP_COLS), lambda i: (0, i))],
    out_specs=pl.BlockSpec((PREP_COLS, D_PAD), lambda i: (i, 0)),
    out_shape=jax.ShapeDtypeStruct((N_VOCAB_ROWS, D_PAD), jnp.float32),
)


@functools.partial(
    pl.kernel,
    out_type=jax.ShapeDtypeStruct((B_TOTAL, D_PAD), jnp.float32),
    mesh=plsc.VectorSubcoreMesh(core_axis_name="c", subcore_axis_name="s"),
    scratch_types=[
        pltpu.VMEM((B_PER_W,), jnp.int32),
        pltpu.VMEM((NBUF, CHUNK, D_PAD), jnp.float32),
        pltpu.SemaphoreType.DMA((NBUF,)),
        pltpu.SemaphoreType.DMA((NBUF,)),
    ],
)
def _embed_gather(idx_hbm, table_hbm, out_hbm, idx_v, rows_v, gsem, wsem):
    wid = lax.axis_index("s") * 2 + lax.axis_index("c")
    base = wid * B_PER_W

    pltpu.sync_copy(idx_hbm.at[pl.ds(base, B_PER_W)], idx_v)

    def start_gather(b, i):
        pltpu.async_copy(
            table_hbm.at[idx_v.at[pl.ds(i * CHUNK, CHUNK)]],
            rows_v.at[b],
            gsem.at[b],
        )

    def wait_gather(b, i):
        pltpu.make_async_copy(
            table_hbm.at[idx_v.at[pl.ds(i * CHUNK, CHUNK)]],
            rows_v.at[b],
            gsem.at[b],
        ).wait()

    def start_wb(b, i):
        pltpu.async_copy(
            rows_v.at[b],
            out_hbm.at[pl.ds(base + i * CHUNK, CHUNK)],
            wsem.at[b],
        )

    def wait_wb(b, i):
        pltpu.make_async_copy(
            rows_v.at[b],
            out_hbm.at[pl.ds(base + i * CHUNK, CHUNK)],
            wsem.at[b],
        ).wait()

    for b in range(NBUF):
        start_gather(b, b)

    def outer(g, carry):
        for b in range(NBUF):
            i = g * NBUF + b
            wait_gather(b, i)
            start_wb(b, i)
            wait_wb(b, i)
            start_gather(b, i + NBUF)
        return carry

    lax.fori_loop(0, NG - 1, outer, 0)

    for b in range(NBUF):
        i = (NG - 1) * NBUF + b
        wait_gather(b, i)
        start_wb(b, i)
    for b in range(NBUF):
        i = (NG - 1) * NBUF + b
        wait_wb(b, i)


def kernel(x, W_E):
    flat = x.reshape(B_TOTAL).astype(jnp.int32)
    table = _prep(W_E.T)
    out = _embed_gather(flat, table)
    return out[:, :D_EMBED].reshape(x.shape[0], x.shape[1], D_EMBED)


# speedup vs baseline: 1.0333x; 1.0060x over previous
import functools

import jax
import jax.numpy as jnp
from jax import lax
from jax.experimental import pallas as pl
from jax.experimental.pallas import tpu as pltpu
from jax.experimental.pallas import tpu_sc as plsc

N_VOCAB_ROWS = 1000000
D_EMBED = 64
D_PAD = 128
B_TOTAL = 4096 * 200
NUM_WORKERS = 32
B_PER_W = B_TOTAL // NUM_WORKERS
CHUNK = 128
N_CHUNK = B_PER_W // CHUNK
NBUF = 5
NG = N_CHUNK // NBUF
assert N_CHUNK % NBUF == 0 and B_PER_W % CHUNK == 0 and B_TOTAL % NUM_WORKERS == 0

PREP_COLS = 32768


def _prep_body(wt_ref, out_ref):
    out_ref[:, 0:D_EMBED] = jnp.transpose(wt_ref[...], (1, 0))


_prep = pl.pallas_call(
    _prep_body,
    grid=(pl.cdiv(N_VOCAB_ROWS, PREP_COLS),),
    in_specs=[pl.BlockSpec((D_EMBED, PREP_COLS), lambda i: (0, i))],
    out_specs=pl.BlockSpec((PREP_COLS, D_PAD), lambda i: (i, 0)),
    out_shape=jax.ShapeDtypeStruct((N_VOCAB_ROWS, D_PAD), jnp.float32),
)


@functools.partial(
    pl.kernel,
    out_type=jax.ShapeDtypeStruct((B_TOTAL, D_PAD), jnp.float32),
    mesh=plsc.VectorSubcoreMesh(core_axis_name="c", subcore_axis_name="s"),
    scratch_types=[
        pltpu.VMEM((B_PER_W,), jnp.int32),
        pltpu.VMEM((NBUF, CHUNK, D_PAD), jnp.float32),
        pltpu.SemaphoreType.DMA((NBUF,)),
        pltpu.SemaphoreType.DMA((NBUF,)),
    ],
)
def _embed_gather(idx_hbm, table_hbm, out_hbm, idx_v, rows_v, gsem, wsem):
    wid = lax.axis_index("s") * 2 + lax.axis_index("c")
    base = wid * B_PER_W

    pltpu.sync_copy(idx_hbm.at[pl.ds(base, B_PER_W)], idx_v)

    def start_gather(b, i):
        pltpu.async_copy(
            table_hbm.at[idx_v.at[pl.ds(i * CHUNK, CHUNK)]],
            rows_v.at[b],
            gsem.at[b],
        )

    def wait_gather(b, i):
        pltpu.make_async_copy(
            table_hbm.at[idx_v.at[pl.ds(i * CHUNK, CHUNK)]],
            rows_v.at[b],
            gsem.at[b],
        ).wait()

    def start_wb(b, i):
        pltpu.async_copy(
            rows_v.at[b],
            out_hbm.at[pl.ds(base + i * CHUNK, CHUNK)],
            wsem.at[b],
        )

    def wait_wb(b, i):
        pltpu.make_async_copy(
            rows_v.at[b],
            out_hbm.at[pl.ds(base + i * CHUNK, CHUNK)],
            wsem.at[b],
        ).wait()

    for b in range(NBUF):
        start_gather(b, b)

    def outer(g, carry):
        for b in range(NBUF):
            i = g * NBUF + b
            wait_gather(b, i)
            start_wb(b, i)
            wait_wb(b, i)
            start_gather(b, i + NBUF)
        return carry

    lax.fori_loop(0, NG - 1, outer, 0)

    for b in range(NBUF):
        i = (NG - 1) * NBUF + b
        wait_gather(b, i)
        start_wb(b, i)
    for b in range(NBUF):
        i = (NG - 1) * NBUF + b
        wait_wb(b, i)


def kernel(x, W_E):
    flat = x.reshape(B_TOTAL).astype(jnp.int32)
    table = _prep(W_E.T)
    out = _embed_gather(flat, table)
    return out[:, :D_EMBED].reshape(x.shape[0], x.shape[1], D_EMBED)
